# Initial kernel scaffold; baseline (speedup 1.0000x reference)
import jax
import jax.numpy as jnp
from jax.experimental import pallas as pl


def kernel(x, edge_index, Wl1, bl1, Wr1, Wl2, bl2, Wr2, Wl3, bl3, Wr3, Wl4, bl4, Wr4, W5, b5, W6, b6, W7, b7):
    raise NotImplementedError("write your pallas kernel here")


# speedup vs baseline: 5.7834x; 5.7834x over previous
import functools

import jax
import jax.numpy as jnp
import numpy as np
from jax import lax
from jax.experimental import pallas as pl
from jax.experimental.pallas import tpu as pltpu
from jax.experimental.pallas import tpu_sc as plsc

N_NODES = 100000
N_EDGES = 1600000
BLK = 1024
NDMA = BLK // 128
NBLK = 1600
E_PAD = NBLK * BLK
ACC_ROWS = 100096
DUMMY_ROW = N_NODES
ZCHUNK = ACC_ROWS // 16
OCHUNK = 6248
ROWS_B = 2000
NGRID = N_NODES // ROWS_B

_MESH = plsc.VectorSubcoreMesh(core_axis_name="c", subcore_axis_name="s")
_SC_PARAMS = pltpu.CompilerParams(use_tc_tiling_on_sc=False)


def _edge_pass(table, out_hbm, out_row0, accum, zeros, src2, dst2,
               src_v, dst_v, rows, gsem, ssem, t, gbase, niter):
    pltpu.sync_copy(zeros.at[pl.ds(t * ZCHUNK, ZCHUNK)],
                    accum.at[pl.ds(t * ZCHUNK, ZCHUNK)])
    plsc.subcore_barrier()

    def blk(i, carry):
        g = gbase + i * 16 + t
        r0 = g * NDMA
        pltpu.sync_copy(src2.at[pl.ds(r0, NDMA)], src_v)
        pltpu.sync_copy(dst2.at[pl.ds(r0, NDMA)], dst_v)
        gh = [pltpu.async_copy(table.at[src_v.at[j]],
                               rows.at[pl.ds(j * 128, 128)], gsem)
              for j in range(NDMA)]
        for h in gh:
            h.wait()
        sh = [pltpu.async_copy(rows.at[pl.ds(j * 128, 128)],
                               accum.at[dst_v.at[j]], ssem, add=True)
              for j in range(NDMA)]
        for h in sh:
            h.wait()
        return carry

    lax.fori_loop(0, niter, blk, 0)
    plsc.subcore_barrier()
    pltpu.sync_copy(accum.at[pl.ds(t * OCHUNK, OCHUNK)],
                    out_hbm.at[pl.ds(out_row0 + t * OCHUNK, OCHUNK)])

    @pl.when(t == 0)
    def _tail():
        base = 16 * OCHUNK
        pltpu.sync_copy(accum.at[pl.ds(base, N_NODES - base)],
                        out_hbm.at[pl.ds(out_row0 + base, N_NODES - base)])

    plsc.subcore_barrier()


@functools.partial(
    pl.kernel,
    mesh=_MESH,
    compiler_params=_SC_PARAMS,
    out_type=jax.ShapeDtypeStruct((2 * N_NODES, 16), jnp.float32),
    scratch_types=[
        pltpu.VMEM((NDMA, 128), jnp.int32),
        pltpu.VMEM((NDMA, 128), jnp.int32),
        pltpu.VMEM((BLK, 16), jnp.float32),
        pltpu.VMEM_SHARED((ACC_ROWS, 16), jnp.float32),
        pltpu.SemaphoreType.DMA,
        pltpu.SemaphoreType.DMA,
    ],
)
def _agg_first(h0p, src2, dst2, zeros, out,
               src_v, dst_v, rows, accum, gsem, ssem):
    c = lax.axis_index("c")
    t = lax.axis_index("s")
    _edge_pass(h0p, out, c * N_NODES, accum, zeros, src2, dst2,
               src_v, dst_v, rows, gsem, ssem, t,
               gbase=c * (NBLK // 2), niter=NBLK // 32)


@functools.partial(
    pl.kernel,
    mesh=_MESH,
    compiler_params=_SC_PARAMS,
    out_type=jax.ShapeDtypeStruct((4 * N_NODES, 16), jnp.float32),
    scratch_types=[
        pltpu.VMEM((NDMA, 128), jnp.int32),
        pltpu.VMEM((NDMA, 128), jnp.int32),
        pltpu.VMEM((BLK, 16), jnp.float32),
        pltpu.VMEM_SHARED((ACC_ROWS, 16), jnp.float32),
        pltpu.SemaphoreType.DMA,
        pltpu.SemaphoreType.DMA,
    ],
)
def _agg_hidden(h0, h1, h2, h3, src2, dst2, zeros, out,
                src_v, dst_v, rows, accum, gsem, ssem):
    c = lax.axis_index("c")
    t = lax.axis_index("s")
    tables = [h0, h1, h2, h3]
    for cc in (0, 1):
        def _branch(cc=cc):
            for qi in (0, 1):
                q = 2 * cc + qi
                _edge_pass(tables[q], out, q * N_NODES, accum, zeros,
                           src2, dst2, src_v, dst_v, rows, gsem, ssem, t,
                           gbase=0, niter=NBLK // 16)
        pl.when(c == cc)(_branch)


def _tc_first_body(s1a, s1b, h0p, wl, bl, wr, o0, o1, o2, o3, orcp):
    s = s1a[...] + s1b[...]
    cnt = s[:, 3:4]
    rcp = 1.0 / jnp.maximum(cnt, 1.0)
    mean = s * rcp
    z = jnp.tanh(jnp.dot(mean, wl[...]) + jnp.dot(h0p[...], wr[...]) + bl[...])
    o0[...] = z[:, 0:16]
    o1[...] = z[:, 16:32]
    o2[...] = z[:, 32:48]
    o3[...] = z[:, 48:64]
    orcp[...] = rcp


def _tc_mid_body(s0, s1, s2, s3, rcp, h0, h1, h2, h3, wl, bl, wr,
                 o0, o1, o2, o3):
    s = jnp.concatenate([s0[...], s1[...], s2[...], s3[...]], axis=1)
    h = jnp.concatenate([h0[...], h1[...], h2[...], h3[...]], axis=1)
    mean = s * rcp[...]
    z = jnp.tanh(jnp.dot(mean, wl[...]) + jnp.dot(h, wr[...]) + bl[...])
    o0[...] = z[:, 0:16]
    o1[...] = z[:, 16:32]
    o2[...] = z[:, 32:48]
    o3[...] = z[:, 48:64]


def _tc_head_body(h0, h1, h2, h3, w5, b5, w6, b6, w7, b7, out):
    h = jnp.concatenate([h0[...], h1[...], h2[...], h3[...]], axis=1)
    z = jnp.tanh(jnp.dot(h, w5[...]) + b5[...])
    z = jnp.tanh(jnp.dot(z, w6[...]) + b6[...])
    z = jnp.dot(z, w7[...]) + b7[...]
    m = jnp.max(z, axis=1, keepdims=True)
    e = jnp.exp(z - m)
    out[...] = e / jnp.sum(e, axis=1, keepdims=True)


def _blk(r, cdim, imap):
    return pl.BlockSpec((r, cdim), imap)


def _full(shape):
    return pl.BlockSpec(shape, lambda i: (0, 0))


def _tc_first(s1, h0p, wl, bl, wr):
    return pl.pallas_call(
        _tc_first_body,
        grid=(NGRID,),
        in_specs=[
            _blk(ROWS_B, 16, lambda i: (i, 0)),
            _blk(ROWS_B, 16, lambda i: (NGRID + i, 0)),
            _blk(ROWS_B, 16, lambda i: (i, 0)),
            _full((16, 64)),
            _full((1, 64)),
            _full((16, 64)),
        ],
        out_specs=[_blk(ROWS_B, 16, lambda i: (i, 0))] * 4
        + [_blk(ROWS_B, 1, lambda i: (i, 0))],
        out_shape=[jax.ShapeDtypeStruct((N_NODES, 16), jnp.float32)] * 4
        + [jax.ShapeDtypeStruct((N_NODES, 1), jnp.float32)],
    )(s1, s1, h0p, wl, bl, wr)


def _tc_mid(s4, rcp, hq, wl, bl, wr):
    def smap(q):
        return _blk(ROWS_B, 16, lambda i, q=q: (q * NGRID + i, 0))

    return pl.pallas_call(
        _tc_mid_body,
        grid=(NGRID,),
        in_specs=[smap(0), smap(1), smap(2), smap(3),
                  _blk(ROWS_B, 1, lambda i: (i, 0))]
        + [_blk(ROWS_B, 16, lambda i: (i, 0))] * 4
        + [_full((64, 64)), _full((1, 64)), _full((64, 64))],
        out_specs=[_blk(ROWS_B, 16, lambda i: (i, 0))] * 4,
        out_shape=[jax.ShapeDtypeStruct((N_NODES, 16), jnp.float32)] * 4,
    )(s4, s4, s4, s4, rcp, *hq, wl, bl, wr)


def _tc_head(hq, w5, b5, w6, b6, w7, b7):
    return pl.pallas_call(
        _tc_head_body,
        grid=(NGRID,),
        in_specs=[_blk(ROWS_B, 16, lambda i: (i, 0))] * 4
        + [_full((64, 64)), _full((1, 64)), _full((64, 64)),
           _full((1, 64)), _full((64, 16)), _full((1, 16))],
        out_specs=_blk(ROWS_B, 16, lambda i: (i, 0)),
        out_shape=jax.ShapeDtypeStruct((N_NODES, 16), jnp.float32),
    )(*hq, w5, b5, w6, b6, w7, b7)


def kernel(x, edge_index, Wl1, bl1, Wr1, Wl2, bl2, Wr2, Wl3, bl3, Wr3,
           Wl4, bl4, Wr4, W5, b5, W6, b6, W7, b7):
    coords = x[:, :2]
    areas = x[:, -1:]
    max_c = jnp.max(coords, axis=0)
    min_c = jnp.min(coords, axis=0)
    rotate = (max_c[1] - min_c[1]) > (max_c[0] - min_c[0])
    theta = jnp.asarray(np.pi / 2, dtype=x.dtype)
    rot = jnp.array([[jnp.cos(theta), -jnp.sin(theta)],
                     [jnp.sin(theta), jnp.cos(theta)]], dtype=x.dtype)
    coords_rot = (rot @ coords.T).T
    coords = jnp.where(rotate, coords_rot, coords)
    coords = (coords - jnp.mean(coords, axis=0)) / jnp.max(coords, axis=0)
    areas = areas / jnp.max(areas, axis=0)

    h0p = jnp.concatenate(
        [coords, areas,
         jnp.ones((N_NODES, 1), jnp.float32),
         jnp.zeros((N_NODES, 12), jnp.float32)], axis=1)

    pad = E_PAD - N_EDGES
    src2 = jnp.concatenate(
        [edge_index[0], jnp.zeros((pad,), jnp.int32)]).reshape(E_PAD // 128, 128)
    dst2 = jnp.concatenate(
        [edge_index[1], jnp.full((pad,), DUMMY_ROW, jnp.int32)]
    ).reshape(E_PAD // 128, 128)
    zeros = jnp.zeros((ACC_ROWS, 16), jnp.float32)

    def padt(w):
        return jnp.pad(w, ((0, 0), (0, 13))).T

    wl1, wr1 = padt(Wl1), padt(Wr1)
    row = lambda b: b.reshape(1, -1)

    s1 = _agg_first(h0p, src2, dst2, zeros)
    o0, o1, o2, o3, rcp = _tc_first(s1, h0p, wl1, row(bl1), wr1)
    hq = [o0, o1, o2, o3]

    for wl, bl, wr in ((Wl2, bl2, Wr2), (Wl3, bl3, Wr3), (Wl4, bl4, Wr4)):
        s4 = _agg_hidden(*hq, src2, dst2, zeros)
        hq = list(_tc_mid(s4, rcp, hq, wl.T, row(bl), wr.T))

    return _tc_head(hq, W5.T, row(b5), W6.T, row(b6), W7.T, row(b7))


# speedup vs baseline: 6.5368x; 1.1303x over previous
import functools

import jax
import jax.numpy as jnp
import numpy as np
from jax import lax
from jax.experimental import pallas as pl
from jax.experimental.pallas import tpu as pltpu
from jax.experimental.pallas import tpu_sc as plsc

N_NODES = 100000
N_EDGES = 1600000
HALF = 512
HD = HALF // 128
E_PAD = 1638400
ACC_ROWS = 100096
DUMMY_ROW = N_NODES
ZCHUNK = ACC_ROWS // 16
OCHUNK = 6248
ROWS_B = 2000
NGRID = N_NODES // ROWS_B

_MESH = plsc.VectorSubcoreMesh(core_axis_name="c", subcore_axis_name="s")
_SC_PARAMS = pltpu.CompilerParams(use_tc_tiling_on_sc=False)


def _edge_pass(table, out_hbm, out_row0, accum, zeros, src2, dst2,
               src_g, dst_g, rows, isems, gsem, ssems, t, gbase_g,
               ngrp, gh):
    rpg = gh * HD
    pltpu.sync_copy(zeros.at[pl.ds(t * ZCHUNK, ZCHUNK)],
                    accum.at[pl.ds(t * ZCHUNK, ZCHUNK)])
    plsc.subcore_barrier()

    def rowbase(j):
        return (gbase_g + j * 16 + t) * rpg

    def drain_idx(p):
        pltpu.make_async_copy(src2.at[pl.ds(0, rpg)], src_g.at[p],
                              isems[p]).wait()
        pltpu.make_async_copy(dst2.at[pl.ds(0, rpg)], dst_g.at[p],
                              isems[p]).wait()

    def drain_scat(b):
        pltpu.make_async_copy(zeros.at[pl.ds(0, HALF)],
                              rows.at[pl.ds(b * HALF, HALF)],
                              ssems[b]).wait()

    def fetch_idx(j, p):
        rb = rowbase(jnp.minimum(j, ngrp - 1))
        pltpu.async_copy(src2.at[pl.ds(rb, rpg)], src_g.at[p], isems[p])
        pltpu.async_copy(dst2.at[pl.ds(rb, rpg)], dst_g.at[p], isems[p])

    fetch_idx(0, 0)

    @pl.loop(0, ngrp, step=2)
    def _grp(jbase):
        for jj in range(2):
            j = jbase + jj
            p = jj
            drain_idx(p)
            for h in range(gh):
                b = h % 2
                if h < 2:
                    @pl.when(j > 0)
                    def _d(b=b):
                        drain_scat(b)
                else:
                    drain_scat(b)
                gts = [pltpu.async_copy(
                    table.at[src_g.at[p, h * HD + u]],
                    rows.at[pl.ds(b * HALF + u * 128, 128)], gsem)
                    for u in range(HD)]
                if h == 1 or gh == 1:
                    fetch_idx(j + 1, 1 - p)
                for g in gts:
                    g.wait()
                for u in range(HD):
                    pltpu.async_copy(
                        rows.at[pl.ds(b * HALF + u * 128, 128)],
                        accum.at[dst_g.at[p, h * HD + u]],
                        ssems[b], add=True)

    drain_scat(0)
    if gh >= 2:
        drain_scat(1)
    drain_idx(0)
    plsc.subcore_barrier()
    pltpu.sync_copy(accum.at[pl.ds(t * OCHUNK, OCHUNK)],
                    out_hbm.at[pl.ds(out_row0 + t * OCHUNK, OCHUNK)])

    @pl.when(t == 0)
    def _tail():
        base = 16 * OCHUNK
        pltpu.sync_copy(accum.at[pl.ds(base, N_NODES - base)],
                        out_hbm.at[pl.ds(out_row0 + base, N_NODES - base)])

    plsc.subcore_barrier()


@functools.partial(
    pl.kernel,
    mesh=_MESH,
    compiler_params=_SC_PARAMS,
    out_type=jax.ShapeDtypeStruct((2 * N_NODES, 16), jnp.float32),
    scratch_types=[
        pltpu.VMEM((2, 2 * HD, 128), jnp.int32),
        pltpu.VMEM((2, 2 * HD, 128), jnp.int32),
        pltpu.VMEM((2 * HALF, 16), jnp.float32),
        pltpu.VMEM_SHARED((ACC_ROWS, 16), jnp.float32),
        pltpu.SemaphoreType.DMA,
        pltpu.SemaphoreType.DMA,
        pltpu.SemaphoreType.DMA,
        pltpu.SemaphoreType.DMA,
        pltpu.SemaphoreType.DMA,
    ],
)
def _agg_first(h0p, src2, dst2, zeros, out,
               src_g, dst_g, rows, accum, isem0, isem1, gsem, ssem0, ssem1):
    c = lax.axis_index("c")
    t = lax.axis_index("s")
    _edge_pass(h0p, out, c * N_NODES, accum, zeros, src2, dst2,
               src_g, dst_g, rows, (isem0, isem1), gsem, (ssem0, ssem1),
               t, gbase_g=c * 800, ngrp=50, gh=2)


@functools.partial(
    pl.kernel,
    mesh=_MESH,
    compiler_params=_SC_PARAMS,
    out_type=jax.ShapeDtypeStruct((4 * N_NODES, 16), jnp.float32),
    scratch_types=[
        pltpu.VMEM((2, 4 * HD, 128), jnp.int32),
        pltpu.VMEM((2, 4 * HD, 128), jnp.int32),
        pltpu.VMEM((2 * HALF, 16), jnp.float32),
        pltpu.VMEM_SHARED((ACC_ROWS, 16), jnp.float32),
        pltpu.SemaphoreType.DMA,
        pltpu.SemaphoreType.DMA,
        pltpu.SemaphoreType.DMA,
        pltpu.SemaphoreType.DMA,
        pltpu.SemaphoreType.DMA,
    ],
)
def _agg_hidden(h0, h1, h2, h3, src2, dst2, zeros, out,
                src_g, dst_g, rows, accum, isem0, isem1, gsem, ssem0, ssem1):
    c = lax.axis_index("c")
    t = lax.axis_index("s")
    tables = [h0, h1, h2, h3]
    for cc in (0, 1):
        def _branch(cc=cc):
            for qi in (0, 1):
                q = 2 * cc + qi
                _edge_pass(tables[q], out, q * N_NODES, accum, zeros,
                           src2, dst2, src_g, dst_g, rows,
                           (isem0, isem1), gsem, (ssem0, ssem1), t,
                           gbase_g=0, ngrp=50, gh=4)
        pl.when(c == cc)(_branch)


def _tc_first_body(s1a, s1b, h0p, wl, bl, wr, o0, o1, o2, o3, orcp):
    s = s1a[...] + s1b[...]
    cnt = s[:, 3:4]
    rcp = 1.0 / jnp.maximum(cnt, 1.0)
    mean = s * rcp
    z = jnp.tanh(jnp.dot(mean, wl[...]) + jnp.dot(h0p[...], wr[...]) + bl[...])
    o0[...] = z[:, 0:16]
    o1[...] = z[:, 16:32]
    o2[...] = z[:, 32:48]
    o3[...] = z[:, 48:64]
    orcp[...] = rcp


def _tc_mid_body(s0, s1, s2, s3, rcp, h0, h1, h2, h3, wl, bl, wr,
                 o0, o1, o2, o3):
    s = jnp.concatenate([s0[...], s1[...], s2[...], s3[...]], axis=1)
    h = jnp.concatenate([h0[...], h1[...], h2[...], h3[...]], axis=1)
    mean = s * rcp[...]
    z = jnp.tanh(jnp.dot(mean, wl[...]) + jnp.dot(h, wr[...]) + bl[...])
    o0[...] = z[:, 0:16]
    o1[...] = z[:, 16:32]
    o2[...] = z[:, 32:48]
    o3[...] = z[:, 48:64]


def _tc_head_body(h0, h1, h2, h3, w5, b5, w6, b6, w7, b7, out):
    h = jnp.concatenate([h0[...], h1[...], h2[...], h3[...]], axis=1)
    z = jnp.tanh(jnp.dot(h, w5[...]) + b5[...])
    z = jnp.tanh(jnp.dot(z, w6[...]) + b6[...])
    z = jnp.dot(z, w7[...]) + b7[...]
    m = jnp.max(z, axis=1, keepdims=True)
    e = jnp.exp(z - m)
    out[...] = e / jnp.sum(e, axis=1, keepdims=True)


def _blk(r, cdim, imap):
    return pl.BlockSpec((r, cdim), imap)


def _full(shape):
    return pl.BlockSpec(shape, lambda i: (0, 0))


def _tc_first(s1, h0p, wl, bl, wr):
    return pl.pallas_call(
        _tc_first_body,
        grid=(NGRID,),
        in_specs=[
            _blk(ROWS_B, 16, lambda i: (i, 0)),
            _blk(ROWS_B, 16, lambda i: (NGRID + i, 0)),
            _blk(ROWS_B, 16, lambda i: (i, 0)),
            _full((16, 64)),
            _full((1, 64)),
            _full((16, 64)),
        ],
        out_specs=[_blk(ROWS_B, 16, lambda i: (i, 0))] * 4
        + [_blk(ROWS_B, 1, lambda i: (i, 0))],
        out_shape=[jax.ShapeDtypeStruct((N_NODES, 16), jnp.float32)] * 4
        + [jax.ShapeDtypeStruct((N_NODES, 1), jnp.float32)],
    )(s1, s1, h0p, wl, bl, wr)


def _tc_mid(s4, rcp, hq, wl, bl, wr):
    def smap(q):
        return _blk(ROWS_B, 16, lambda i, q=q: (q * NGRID + i, 0))

    return pl.pallas_call(
        _tc_mid_body,
        grid=(NGRID,),
        in_specs=[smap(0), smap(1), smap(2), smap(3),
                  _blk(ROWS_B, 1, lambda i: (i, 0))]
        + [_blk(ROWS_B, 16, lambda i: (i, 0))] * 4
        + [_full((64, 64)), _full((1, 64)), _full((64, 64))],
        out_specs=[_blk(ROWS_B, 16, lambda i: (i, 0))] * 4,
        out_shape=[jax.ShapeDtypeStruct((N_NODES, 16), jnp.float32)] * 4,
    )(s4, s4, s4, s4, rcp, *hq, wl, bl, wr)


def _tc_head(hq, w5, b5, w6, b6, w7, b7):
    return pl.pallas_call(
        _tc_head_body,
        grid=(NGRID,),
        in_specs=[_blk(ROWS_B, 16, lambda i: (i, 0))] * 4
        + [_full((64, 64)), _full((1, 64)), _full((64, 64)),
           _full((1, 64)), _full((64, 16)), _full((1, 16))],
        out_specs=_blk(ROWS_B, 16, lambda i: (i, 0)),
        out_shape=jax.ShapeDtypeStruct((N_NODES, 16), jnp.float32),
    )(*hq, w5, b5, w6, b6, w7, b7)


def kernel(x, edge_index, Wl1, bl1, Wr1, Wl2, bl2, Wr2, Wl3, bl3, Wr3,
           Wl4, bl4, Wr4, W5, b5, W6, b6, W7, b7):
    coords = x[:, :2]
    areas = x[:, -1:]
    max_c = jnp.max(coords, axis=0)
    min_c = jnp.min(coords, axis=0)
    rotate = (max_c[1] - min_c[1]) > (max_c[0] - min_c[0])
    theta = jnp.asarray(np.pi / 2, dtype=x.dtype)
    rot = jnp.array([[jnp.cos(theta), -jnp.sin(theta)],
                     [jnp.sin(theta), jnp.cos(theta)]], dtype=x.dtype)
    coords_rot = (rot @ coords.T).T
    coords = jnp.where(rotate, coords_rot, coords)
    coords = (coords - jnp.mean(coords, axis=0)) / jnp.max(coords, axis=0)
    areas = areas / jnp.max(areas, axis=0)

    h0p = jnp.concatenate(
        [coords, areas,
         jnp.ones((N_NODES, 1), jnp.float32),
         jnp.zeros((N_NODES, 12), jnp.float32)], axis=1)

    pad = E_PAD - N_EDGES
    src2 = jnp.concatenate(
        [edge_index[0], jnp.zeros((pad,), jnp.int32)]).reshape(E_PAD // 128, 128)
    dst2 = jnp.concatenate(
        [edge_index[1], jnp.full((pad,), DUMMY_ROW, jnp.int32)]
    ).reshape(E_PAD // 128, 128)
    zeros = jnp.zeros((ACC_ROWS, 16), jnp.float32)

    def padt(w):
        return jnp.pad(w, ((0, 0), (0, 13))).T

    wl1, wr1 = padt(Wl1), padt(Wr1)
    row = lambda b: b.reshape(1, -1)

    s1 = _agg_first(h0p, src2, dst2, zeros)
    o0, o1, o2, o3, rcp = _tc_first(s1, h0p, wl1, row(bl1), wr1)
    hq = [o0, o1, o2, o3]

    for wl, bl, wr in ((Wl2, bl2, Wr2), (Wl3, bl3, Wr3), (Wl4, bl4, Wr4)):
        s4 = _agg_hidden(*hq, src2, dst2, zeros)
        hq = list(_tc_mid(s4, rcp, hq, wl.T, row(bl), wr.T))

    return _tc_head(hq, W5.T, row(b5), W6.T, row(b6), W7.T, row(b7))


# speedup vs baseline: 6.8485x; 1.0477x over previous
import functools

import jax
import jax.numpy as jnp
import numpy as np
from jax import lax
from jax.experimental import pallas as pl
from jax.experimental.pallas import tpu as pltpu
from jax.experimental.pallas import tpu_sc as plsc

N_NODES = 100000
N_EDGES = 1600000
HALF = 512
HD = HALF // 128
E_PAD = 1638400
ACC_ROWS = 100096
DUMMY_ROW = N_NODES
ZCHUNK = ACC_ROWS // 16
OCHUNK = 6248
ROWS_B = 2000
NGRID = N_NODES // ROWS_B

_MESH = plsc.VectorSubcoreMesh(core_axis_name="c", subcore_axis_name="s")
_SC_PARAMS = pltpu.CompilerParams(use_tc_tiling_on_sc=False)


def _edge_pass(table, out_hbm, out_row0, accum, zeros, src2, dst2,
               src_g, dst_g, rows, isems, gsems, ssems, t, gbase_g,
               ngrp, gh):
    rpg = gh * HD
    pltpu.sync_copy(zeros.at[pl.ds(t * ZCHUNK, ZCHUNK)],
                    accum.at[pl.ds(t * ZCHUNK, ZCHUNK)])
    plsc.subcore_barrier()

    def rowbase(j):
        return (gbase_g + j * 16 + t) * rpg

    def drain_idx(p):
        pltpu.make_async_copy(src2.at[pl.ds(0, rpg)], src_g.at[p],
                              isems[p]).wait()
        pltpu.make_async_copy(dst2.at[pl.ds(0, rpg)], dst_g.at[p],
                              isems[p]).wait()

    def drain_half(b, sem):
        pltpu.make_async_copy(zeros.at[pl.ds(0, HALF)],
                              rows.at[pl.ds(b * HALF, HALF)], sem).wait()

    def fetch_idx(j, p):
        rb = rowbase(jnp.minimum(j, ngrp - 1))
        pltpu.async_copy(src2.at[pl.ds(rb, rpg)], src_g.at[p], isems[p])
        pltpu.async_copy(dst2.at[pl.ds(rb, rpg)], dst_g.at[p], isems[p])

    def fire_gathers(p, h, b):
        for u in range(HD):
            pltpu.async_copy(table.at[src_g.at[p, h * HD + u]],
                             rows.at[pl.ds(b * HALF + u * 128, 128)],
                             gsems[b])

    fetch_idx(0, 0)
    drain_idx(0)
    fire_gathers(0, 0, 0)

    @pl.loop(0, ngrp, step=2)
    def _grp(jbase):
        for jj in range(2):
            j = jbase + jj
            p = jj
            for h in range(gh):
                b = h % 2
                if h == 0:
                    @pl.when(j > 0)
                    def _d(b=b):
                        drain_half(1 - b, ssems[1 - b])
                else:
                    drain_half(1 - b, ssems[1 - b])
                if h == 1:
                    fetch_idx(j + 1, 1 - p)
                if h == gh - 1:
                    drain_idx(1 - p)
                    fire_gathers(1 - p, 0, 1 - b)
                else:
                    fire_gathers(p, h + 1, 1 - b)
                drain_half(b, gsems[b])
                for u in range(HD):
                    pltpu.async_copy(
                        rows.at[pl.ds(b * HALF + u * 128, 128)],
                        accum.at[dst_g.at[p, h * HD + u]],
                        ssems[b], add=True)

    last_b = (ngrp * gh - 1) % 2
    drain_half(1 - last_b, gsems[1 - last_b])
    drain_half(last_b, ssems[last_b])
    plsc.subcore_barrier()
    pltpu.sync_copy(accum.at[pl.ds(t * OCHUNK, OCHUNK)],
                    out_hbm.at[pl.ds(out_row0 + t * OCHUNK, OCHUNK)])

    @pl.when(t == 0)
    def _tail():
        base = 16 * OCHUNK
        pltpu.sync_copy(accum.at[pl.ds(base, N_NODES - base)],
                        out_hbm.at[pl.ds(out_row0 + base, N_NODES - base)])

    plsc.subcore_barrier()


@functools.partial(
    pl.kernel,
    mesh=_MESH,
    compiler_params=_SC_PARAMS,
    out_type=jax.ShapeDtypeStruct((2 * N_NODES, 16), jnp.float32),
    scratch_types=[
        pltpu.VMEM((2, 2 * HD, 128), jnp.int32),
        pltpu.VMEM((2, 2 * HD, 128), jnp.int32),
        pltpu.VMEM((2 * HALF, 16), jnp.float32),
        pltpu.VMEM_SHARED((ACC_ROWS, 16), jnp.float32),
        pltpu.SemaphoreType.DMA,
        pltpu.SemaphoreType.DMA,
        pltpu.SemaphoreType.DMA,
        pltpu.SemaphoreType.DMA,
        pltpu.SemaphoreType.DMA,
        pltpu.SemaphoreType.DMA,
    ],
)
def _agg_first(h0p, src2, dst2, zeros, out,
               src_g, dst_g, rows, accum, isem0, isem1, gsem0, gsem1, ssem0, ssem1):
    c = lax.axis_index("c")
    t = lax.axis_index("s")
    _edge_pass(h0p, out, c * N_NODES, accum, zeros, src2, dst2,
               src_g, dst_g, rows, (isem0, isem1), (gsem0, gsem1), (ssem0, ssem1),
               t, gbase_g=c * 800, ngrp=50, gh=2)


@functools.partial(
    pl.kernel,
    mesh=_MESH,
    compiler_params=_SC_PARAMS,
    out_type=jax.ShapeDtypeStruct((4 * N_NODES, 16), jnp.float32),
    scratch_types=[
        pltpu.VMEM((2, 4 * HD, 128), jnp.int32),
        pltpu.VMEM((2, 4 * HD, 128), jnp.int32),
        pltpu.VMEM((2 * HALF, 16), jnp.float32),
        pltpu.VMEM_SHARED((ACC_ROWS, 16), jnp.float32),
        pltpu.SemaphoreType.DMA,
        pltpu.SemaphoreType.DMA,
        pltpu.SemaphoreType.DMA,
        pltpu.SemaphoreType.DMA,
        pltpu.SemaphoreType.DMA,
        pltpu.SemaphoreType.DMA,
    ],
)
def _agg_hidden(h0, h1, h2, h3, src2, dst2, zeros, out,
                src_g, dst_g, rows, accum, isem0, isem1, gsem0, gsem1, ssem0, ssem1):
    c = lax.axis_index("c")
    t = lax.axis_index("s")
    tables = [h0, h1, h2, h3]
    for cc in (0, 1):
        def _branch(cc=cc):
            for qi in (0, 1):
                q = 2 * cc + qi
                _edge_pass(tables[q], out, q * N_NODES, accum, zeros,
                           src2, dst2, src_g, dst_g, rows,
                           (isem0, isem1), (gsem0, gsem1), (ssem0, ssem1), t,
                           gbase_g=0, ngrp=50, gh=4)
        pl.when(c == cc)(_branch)


def _tc_first_body(s1a, s1b, h0p, wl, bl, wr, o0, o1, o2, o3, orcp):
    s = s1a[...] + s1b[...]
    cnt = s[:, 3:4]
    rcp = 1.0 / jnp.maximum(cnt, 1.0)
    mean = s * rcp
    z = jnp.tanh(jnp.dot(mean, wl[...]) + jnp.dot(h0p[...], wr[...]) + bl[...])
    o0[...] = z[:, 0:16]
    o1[...] = z[:, 16:32]
    o2[...] = z[:, 32:48]
    o3[...] = z[:, 48:64]
    orcp[...] = rcp


def _tc_mid_body(s0, s1, s2, s3, rcp, h0, h1, h2, h3, wl, bl, wr,
                 o0, o1, o2, o3):
    s = jnp.concatenate([s0[...], s1[...], s2[...], s3[...]], axis=1)
    h = jnp.concatenate([h0[...], h1[...], h2[...], h3[...]], axis=1)
    mean = s * rcp[...]
    z = jnp.tanh(jnp.dot(mean, wl[...]) + jnp.dot(h, wr[...]) + bl[...])
    o0[...] = z[:, 0:16]
    o1[...] = z[:, 16:32]
    o2[...] = z[:, 32:48]
    o3[...] = z[:, 48:64]


def _tc_head_body(h0, h1, h2, h3, w5, b5, w6, b6, w7, b7, out):
    h = jnp.concatenate([h0[...], h1[...], h2[...], h3[...]], axis=1)
    z = jnp.tanh(jnp.dot(h, w5[...]) + b5[...])
    z = jnp.tanh(jnp.dot(z, w6[...]) + b6[...])
    z = jnp.dot(z, w7[...]) + b7[...]
    m = jnp.max(z, axis=1, keepdims=True)
    e = jnp.exp(z - m)
    out[...] = e / jnp.sum(e, axis=1, keepdims=True)


def _blk(r, cdim, imap):
    return pl.BlockSpec((r, cdim), imap)


def _full(shape):
    return pl.BlockSpec(shape, lambda i: (0, 0))


def _tc_first(s1, h0p, wl, bl, wr):
    return pl.pallas_call(
        _tc_first_body,
        grid=(NGRID,),
        in_specs=[
            _blk(ROWS_B, 16, lambda i: (i, 0)),
            _blk(ROWS_B, 16, lambda i: (NGRID + i, 0)),
            _blk(ROWS_B, 16, lambda i: (i, 0)),
            _full((16, 64)),
            _full((1, 64)),
            _full((16, 64)),
        ],
        out_specs=[_blk(ROWS_B, 16, lambda i: (i, 0))] * 4
        + [_blk(ROWS_B, 1, lambda i: (i, 0))],
        out_shape=[jax.ShapeDtypeStruct((N_NODES, 16), jnp.float32)] * 4
        + [jax.ShapeDtypeStruct((N_NODES, 1), jnp.float32)],
    )(s1, s1, h0p, wl, bl, wr)


def _tc_mid(s4, rcp, hq, wl, bl, wr):
    def smap(q):
        return _blk(ROWS_B, 16, lambda i, q=q: (q * NGRID + i, 0))

    return pl.pallas_call(
        _tc_mid_body,
        grid=(NGRID,),
        in_specs=[smap(0), smap(1), smap(2), smap(3),
                  _blk(ROWS_B, 1, lambda i: (i, 0))]
        + [_blk(ROWS_B, 16, lambda i: (i, 0))] * 4
        + [_full((64, 64)), _full((1, 64)), _full((64, 64))],
        out_specs=[_blk(ROWS_B, 16, lambda i: (i, 0))] * 4,
        out_shape=[jax.ShapeDtypeStruct((N_NODES, 16), jnp.float32)] * 4,
    )(s4, s4, s4, s4, rcp, *hq, wl, bl, wr)


def _tc_head(hq, w5, b5, w6, b6, w7, b7):
    return pl.pallas_call(
        _tc_head_body,
        grid=(NGRID,),
        in_specs=[_blk(ROWS_B, 16, lambda i: (i, 0))] * 4
        + [_full((64, 64)), _full((1, 64)), _full((64, 64)),
           _full((1, 64)), _full((64, 16)), _full((1, 16))],
        out_specs=_blk(ROWS_B, 16, lambda i: (i, 0)),
        out_shape=jax.ShapeDtypeStruct((N_NODES, 16), jnp.float32),
    )(*hq, w5, b5, w6, b6, w7, b7)


def kernel(x, edge_index, Wl1, bl1, Wr1, Wl2, bl2, Wr2, Wl3, bl3, Wr3,
           Wl4, bl4, Wr4, W5, b5, W6, b6, W7, b7):
    coords = x[:, :2]
    areas = x[:, -1:]
    max_c = jnp.max(coords, axis=0)
    min_c = jnp.min(coords, axis=0)
    rotate = (max_c[1] - min_c[1]) > (max_c[0] - min_c[0])
    theta = jnp.asarray(np.pi / 2, dtype=x.dtype)
    rot = jnp.array([[jnp.cos(theta), -jnp.sin(theta)],
                     [jnp.sin(theta), jnp.cos(theta)]], dtype=x.dtype)
    coords_rot = (rot @ coords.T).T
    coords = jnp.where(rotate, coords_rot, coords)
    coords = (coords - jnp.mean(coords, axis=0)) / jnp.max(coords, axis=0)
    areas = areas / jnp.max(areas, axis=0)

    h0p = jnp.concatenate(
        [coords, areas,
         jnp.ones((N_NODES, 1), jnp.float32),
         jnp.zeros((N_NODES, 12), jnp.float32)], axis=1)

    pad = E_PAD - N_EDGES
    src2 = jnp.concatenate(
        [edge_index[0], jnp.zeros((pad,), jnp.int32)]).reshape(E_PAD // 128, 128)
    dst2 = jnp.concatenate(
        [edge_index[1], jnp.full((pad,), DUMMY_ROW, jnp.int32)]
    ).reshape(E_PAD // 128, 128)
    zeros = jnp.zeros((ACC_ROWS, 16), jnp.float32)

    def padt(w):
        return jnp.pad(w, ((0, 0), (0, 13))).T

    wl1, wr1 = padt(Wl1), padt(Wr1)
    row = lambda b: b.reshape(1, -1)

    s1 = _agg_first(h0p, src2, dst2, zeros)
    o0, o1, o2, o3, rcp = _tc_first(s1, h0p, wl1, row(bl1), wr1)
    hq = [o0, o1, o2, o3]

    for wl, bl, wr in ((Wl2, bl2, Wr2), (Wl3, bl3, Wr3), (Wl4, bl4, Wr4)):
        s4 = _agg_hidden(*hq, src2, dst2, zeros)
        hq = list(_tc_mid(s4, rcp, hq, wl.T, row(bl), wr.T))

    return _tc_head(hq, W5.T, row(b5), W6.T, row(b6), W7.T, row(b7))


# speedup vs baseline: 6.9198x; 1.0104x over previous
import functools

import jax
import jax.numpy as jnp
import numpy as np
from jax import lax
from jax.experimental import pallas as pl
from jax.experimental.pallas import tpu as pltpu
from jax.experimental.pallas import tpu_sc as plsc

N_NODES = 100000
N_EDGES = 1600000
HALF = 512
HD = HALF // 128
E_PAD = 1638400
ACC_ROWS = 100096
DUMMY_ROW = N_NODES
ZCHUNK = ACC_ROWS // 16
OCHUNK = 6248
ROWS_B = 4000
NGRID = N_NODES // ROWS_B

_MESH = plsc.VectorSubcoreMesh(core_axis_name="c", subcore_axis_name="s")
_SC_PARAMS = pltpu.CompilerParams(use_tc_tiling_on_sc=False)


def _edge_pass(table, out_hbm, out_row0, accum, zeros, src2, dst2,
               src_g, dst_g, rows, isems, gsems, ssems, t, gbase_g,
               ngrp, gh):
    rpg = gh * HD
    pltpu.sync_copy(zeros.at[pl.ds(t * ZCHUNK, ZCHUNK)],
                    accum.at[pl.ds(t * ZCHUNK, ZCHUNK)])
    plsc.subcore_barrier()

    def rowbase(j):
        return (gbase_g + j * 16 + t) * rpg

    def drain_idx(p):
        pltpu.make_async_copy(src2.at[pl.ds(0, rpg)], src_g.at[p],
                              isems[p]).wait()
        pltpu.make_async_copy(dst2.at[pl.ds(0, rpg)], dst_g.at[p],
                              isems[p]).wait()

    def drain_half(b, sem):
        pltpu.make_async_copy(zeros.at[pl.ds(0, HALF)],
                              rows.at[pl.ds(b * HALF, HALF)], sem).wait()

    def fetch_idx(j, p):
        rb = rowbase(jnp.minimum(j, ngrp - 1))
        pltpu.async_copy(src2.at[pl.ds(rb, rpg)], src_g.at[p], isems[p])
        pltpu.async_copy(dst2.at[pl.ds(rb, rpg)], dst_g.at[p], isems[p])

    def fire_gathers(p, h, b):
        for u in range(HD):
            pltpu.async_copy(table.at[src_g.at[p, h * HD + u]],
                             rows.at[pl.ds(b * HALF + u * 128, 128)],
                             gsems[b])

    fetch_idx(0, 0)
    drain_idx(0)
    fire_gathers(0, 0, 0)

    @pl.loop(0, ngrp, step=2)
    def _grp(jbase):
        for jj in range(2):
            j = jbase + jj
            p = jj
            for h in range(gh):
                b = h % 2
                if h == 0:
                    @pl.when(j > 0)
                    def _d(b=b):
                        drain_half(1 - b, ssems[1 - b])
                else:
                    drain_half(1 - b, ssems[1 - b])
                if h == 1:
                    fetch_idx(j + 1, 1 - p)
                if h == gh - 1:
                    drain_idx(1 - p)
                    fire_gathers(1 - p, 0, 1 - b)
                else:
                    fire_gathers(p, h + 1, 1 - b)
                drain_half(b, gsems[b])
                for u in range(HD):
                    pltpu.async_copy(
                        rows.at[pl.ds(b * HALF + u * 128, 128)],
                        accum.at[dst_g.at[p, h * HD + u]],
                        ssems[b], add=True)

    last_b = (ngrp * gh - 1) % 2
    drain_half(1 - last_b, gsems[1 - last_b])
    drain_half(last_b, ssems[last_b])
    plsc.subcore_barrier()
    pltpu.sync_copy(accum.at[pl.ds(t * OCHUNK, OCHUNK)],
                    out_hbm.at[pl.ds(out_row0 + t * OCHUNK, OCHUNK)])

    @pl.when(t == 0)
    def _tail():
        base = 16 * OCHUNK
        pltpu.sync_copy(accum.at[pl.ds(base, N_NODES - base)],
                        out_hbm.at[pl.ds(out_row0 + base, N_NODES - base)])

    plsc.subcore_barrier()


@functools.partial(
    pl.kernel,
    mesh=_MESH,
    compiler_params=_SC_PARAMS,
    out_type=jax.ShapeDtypeStruct((2 * N_NODES, 16), jnp.float32),
    scratch_types=[
        pltpu.VMEM((2, 2 * HD, 128), jnp.int32),
        pltpu.VMEM((2, 2 * HD, 128), jnp.int32),
        pltpu.VMEM((2 * HALF, 16), jnp.float32),
        pltpu.VMEM_SHARED((ACC_ROWS, 16), jnp.float32),
        pltpu.SemaphoreType.DMA,
        pltpu.SemaphoreType.DMA,
        pltpu.SemaphoreType.DMA,
        pltpu.SemaphoreType.DMA,
        pltpu.SemaphoreType.DMA,
        pltpu.SemaphoreType.DMA,
    ],
)
def _agg_first(h0p, src2, dst2, zeros, out,
               src_g, dst_g, rows, accum, isem0, isem1, gsem0, gsem1, ssem0, ssem1):
    c = lax.axis_index("c")
    t = lax.axis_index("s")
    _edge_pass(h0p, out, c * N_NODES, accum, zeros, src2, dst2,
               src_g, dst_g, rows, (isem0, isem1), (gsem0, gsem1), (ssem0, ssem1),
               t, gbase_g=c * 800, ngrp=50, gh=2)


@functools.partial(
    pl.kernel,
    mesh=_MESH,
    compiler_params=_SC_PARAMS,
    out_type=jax.ShapeDtypeStruct((4 * N_NODES, 16), jnp.float32),
    scratch_types=[
        pltpu.VMEM((2, 4 * HD, 128), jnp.int32),
        pltpu.VMEM((2, 4 * HD, 128), jnp.int32),
        pltpu.VMEM((2 * HALF, 16), jnp.float32),
        pltpu.VMEM_SHARED((ACC_ROWS, 16), jnp.float32),
        pltpu.SemaphoreType.DMA,
        pltpu.SemaphoreType.DMA,
        pltpu.SemaphoreType.DMA,
        pltpu.SemaphoreType.DMA,
        pltpu.SemaphoreType.DMA,
        pltpu.SemaphoreType.DMA,
    ],
)
def _agg_hidden(h0, h1, h2, h3, src2, dst2, zeros, out,
                src_g, dst_g, rows, accum, isem0, isem1, gsem0, gsem1, ssem0, ssem1):
    c = lax.axis_index("c")
    t = lax.axis_index("s")
    tables = [h0, h1, h2, h3]
    for cc in (0, 1):
        def _branch(cc=cc):
            for qi in (0, 1):
                q = 2 * cc + qi
                _edge_pass(tables[q], out, q * N_NODES, accum, zeros,
                           src2, dst2, src_g, dst_g, rows,
                           (isem0, isem1), (gsem0, gsem1), (ssem0, ssem1), t,
                           gbase_g=0, ngrp=50, gh=4)
        pl.when(c == cc)(_branch)


def _tc_first_body(s1a, s1b, h0p, wl, bl, wr, o0, o1, o2, o3, orcp):
    s = s1a[...] + s1b[...]
    cnt = s[:, 3:4]
    rcp = 1.0 / jnp.maximum(cnt, 1.0)
    mean = s * rcp
    z = jnp.tanh(jnp.dot(mean, wl[...]) + jnp.dot(h0p[...], wr[...]) + bl[...])
    o0[...] = z[:, 0:16]
    o1[...] = z[:, 16:32]
    o2[...] = z[:, 32:48]
    o3[...] = z[:, 48:64]
    orcp[...] = rcp


def _tc_mid_body(s0, s1, s2, s3, rcp, h0, h1, h2, h3, wl, bl, wr,
                 o0, o1, o2, o3):
    rcp_v = rcp[...]
    z = bl[...] + jnp.zeros((s0.shape[0], 64), jnp.float32)
    for q, (s, h) in enumerate(((s0, h0), (s1, h1), (s2, h2), (s3, h3))):
        z += jnp.dot(s[...] * rcp_v, wl[pl.ds(16 * q, 16), :])
        z += jnp.dot(h[...], wr[pl.ds(16 * q, 16), :])
    z = jnp.tanh(z)
    o0[...] = z[:, 0:16]
    o1[...] = z[:, 16:32]
    o2[...] = z[:, 32:48]
    o3[...] = z[:, 48:64]


def _tc_head_body(h0, h1, h2, h3, w5, b5, w6, b6, w7, b7, out):
    z = b5[...] + jnp.zeros((h0.shape[0], 64), jnp.float32)
    for q, h in enumerate((h0, h1, h2, h3)):
        z += jnp.dot(h[...], w5[pl.ds(16 * q, 16), :])
    z = jnp.tanh(z)
    z = jnp.tanh(jnp.dot(z, w6[...]) + b6[...])
    z = jnp.dot(z, w7[...]) + b7[...]
    m = jnp.max(z, axis=1, keepdims=True)
    e = jnp.exp(z - m)
    out[...] = e / jnp.sum(e, axis=1, keepdims=True)


def _blk(r, cdim, imap):
    return pl.BlockSpec((r, cdim), imap)


def _full(shape):
    return pl.BlockSpec(shape, lambda i: (0, 0))


def _tc_first(s1, h0p, wl, bl, wr):
    return pl.pallas_call(
        _tc_first_body,
        grid=(NGRID,),
        in_specs=[
            _blk(ROWS_B, 16, lambda i: (i, 0)),
            _blk(ROWS_B, 16, lambda i: (NGRID + i, 0)),
            _blk(ROWS_B, 16, lambda i: (i, 0)),
            _full((16, 64)),
            _full((1, 64)),
            _full((16, 64)),
        ],
        out_specs=[_blk(ROWS_B, 16, lambda i: (i, 0))] * 4
        + [_blk(ROWS_B, 1, lambda i: (i, 0))],
        out_shape=[jax.ShapeDtypeStruct((N_NODES, 16), jnp.float32)] * 4
        + [jax.ShapeDtypeStruct((N_NODES, 1), jnp.float32)],
    )(s1, s1, h0p, wl, bl, wr)


def _tc_mid(s4, rcp, hq, wl, bl, wr):
    def smap(q):
        return _blk(ROWS_B, 16, lambda i, q=q: (q * NGRID + i, 0))

    return pl.pallas_call(
        _tc_mid_body,
        grid=(NGRID,),
        in_specs=[smap(0), smap(1), smap(2), smap(3),
                  _blk(ROWS_B, 1, lambda i: (i, 0))]
        + [_blk(ROWS_B, 16, lambda i: (i, 0))] * 4
        + [_full((64, 64)), _full((1, 64)), _full((64, 64))],
        out_specs=[_blk(ROWS_B, 16, lambda i: (i, 0))] * 4,
        out_shape=[jax.ShapeDtypeStruct((N_NODES, 16), jnp.float32)] * 4,
    )(s4, s4, s4, s4, rcp, *hq, wl, bl, wr)


def _tc_head(hq, w5, b5, w6, b6, w7, b7):
    return pl.pallas_call(
        _tc_head_body,
        grid=(NGRID,),
        in_specs=[_blk(ROWS_B, 16, lambda i: (i, 0))] * 4
        + [_full((64, 64)), _full((1, 64)), _full((64, 64)),
           _full((1, 64)), _full((64, 16)), _full((1, 16))],
        out_specs=_blk(ROWS_B, 16, lambda i: (i, 0)),
        out_shape=jax.ShapeDtypeStruct((N_NODES, 16), jnp.float32),
    )(*hq, w5, b5, w6, b6, w7, b7)


def kernel(x, edge_index, Wl1, bl1, Wr1, Wl2, bl2, Wr2, Wl3, bl3, Wr3,
           Wl4, bl4, Wr4, W5, b5, W6, b6, W7, b7):
    coords = x[:, :2]
    areas = x[:, -1:]
    max_c = jnp.max(coords, axis=0)
    min_c = jnp.min(coords, axis=0)
    rotate = (max_c[1] - min_c[1]) > (max_c[0] - min_c[0])
    theta = jnp.asarray(np.pi / 2, dtype=x.dtype)
    rot = jnp.array([[jnp.cos(theta), -jnp.sin(theta)],
                     [jnp.sin(theta), jnp.cos(theta)]], dtype=x.dtype)
    coords_rot = (rot @ coords.T).T
    coords = jnp.where(rotate, coords_rot, coords)
    coords = (coords - jnp.mean(coords, axis=0)) / jnp.max(coords, axis=0)
    areas = areas / jnp.max(areas, axis=0)

    h0p = jnp.concatenate(
        [coords, areas,
         jnp.ones((N_NODES, 1), jnp.float32),
         jnp.zeros((N_NODES, 12), jnp.float32)], axis=1)

    pad = E_PAD - N_EDGES
    src2 = jnp.concatenate(
        [edge_index[0], jnp.zeros((pad,), jnp.int32)]).reshape(E_PAD // 128, 128)
    dst2 = jnp.concatenate(
        [edge_index[1], jnp.full((pad,), DUMMY_ROW, jnp.int32)]
    ).reshape(E_PAD // 128, 128)
    zeros = jnp.zeros((ACC_ROWS, 16), jnp.float32)

    def padt(w):
        return jnp.pad(w, ((0, 0), (0, 13))).T

    wl1, wr1 = padt(Wl1), padt(Wr1)
    row = lambda b: b.reshape(1, -1)

    s1 = _agg_first(h0p, src2, dst2, zeros)
    o0, o1, o2, o3, rcp = _tc_first(s1, h0p, wl1, row(bl1), wr1)
    hq = [o0, o1, o2, o3]

    for wl, bl, wr in ((Wl2, bl2, Wr2), (Wl3, bl3, Wr3), (Wl4, bl4, Wr4)):
        s4 = _agg_hidden(*hq, src2, dst2, zeros)
        hq = list(_tc_mid(s4, rcp, hq, wl.T, row(bl), wr.T))

    return _tc_head(hq, W5.T, row(b5), W6.T, row(b6), W7.T, row(b7))
